# baseline (speedup 1.0000x reference)
import functools
import jax
import jax.numpy as jnp
from jax import lax
from jax.experimental import pallas as pl
from jax.experimental.pallas import tpu as pltpu
from jax.experimental.pallas import tpu_sc as plsc

C = 768
N = 1024
K = 32
B = 8
NT_PROJ = 256
T_AGG = 128

NW = 32
ROWS = B * N
RPW = ROWS // NW
NCH = N // 16


def _featpos_body(x_ref, w_ref, b_ref, feat_ref, pos_ref):
    xb = x_ref[0]
    w = w_ref[...]
    fp = lax.dot_general(w, xb, (((1,), (0,)), ((), ())),
                         preferred_element_type=jnp.float32)
    fp = fp + b_ref[...]
    feat = fp[:C, :]
    posu = fp[C:, :]
    ss = jnp.sum(posu * posu, axis=0, keepdims=True)
    inv = 1.0 / jnp.clip(jnp.sqrt(ss), 1e-12)
    feat_ref[0] = feat
    pos_ref[0] = posu * inv


def _sim_body(pos_t_ref, pos_ref, sim_ref):
    sim_ref[0] = lax.dot_general(pos_t_ref[0], pos_ref[0],
                                 (((0,), (0,)), ((), ())),
                                 preferred_element_type=jnp.float32)


def _agg_body(sim_ref, thr_ref, feat_ref, out_ref):
    sim = sim_ref[0]
    thr = thr_ref[0, 0]
    mask = sim >= thr[:, None]
    e = jnp.where(mask, jnp.exp(sim - 1.0), 0.0)
    s = jnp.sum(e, axis=1, keepdims=True)
    attn = e / s
    out_ref[0] = lax.dot_general(feat_ref[0], attn, (((1,), (1,)), ((), ())),
                                 preferred_element_type=jnp.float32)


def _kth_sc_body(sim_hbm, thr_hbm, rows_v, u_v, cand_v, hist_v, thru_v,
                 thrf_v, sem, rpw):
    wid = lax.axis_index("s") * 2 + lax.axis_index("c")
    base = wid * rpw
    lanes = lax.iota(jnp.int32, 16)
    ones = jnp.ones((16,), jnp.int32)
    zeros16 = jnp.zeros((16,), jnp.int32)

    def find_bin(k_rem):
        tv = jnp.zeros((16,), jnp.int32)
        for j in range(16):
            tv = jnp.where(lanes == j, jnp.sum(hist_v[pl.ds(j * 16, 16)]), tv)
        rev_tv = lax.rev(tv, (0,))
        cst = plsc.cumsum(rev_tv)
        lane_rev = jnp.max(plsc.all_reduce_ffs(cst >= k_rem))
        jstar = 15 - lane_rev
        accstar = jnp.sum(jnp.where(lanes == lane_rev, cst - rev_tv, 0))
        h = hist_v[pl.ds(jstar * 16, 16)]
        rev = lax.rev(h, (0,))
        cs = plsc.cumsum(rev)
        lane2 = jnp.max(plsc.all_reduce_ffs((accstar + cs) >= k_rem))
        cnt_gt = accstar + jnp.sum(jnp.where(lanes == lane2, cs - rev, 0))
        bbin = jstar * 16 + (15 - lane2)
        return bbin, k_rem - cnt_gt

    pltpu.async_copy(sim_hbm.at[base], rows_v.at[pl.ds(0, N)], sem)

    def row_body(r, pending):
        par = (r % 2) * N

        @pl.when(r + 1 < rpw)
        def _():
            nxt = ((r + 1) % 2) * N
            pltpu.async_copy(sim_hbm.at[base + r + 1],
                             rows_v.at[pl.ds(nxt, N)], sem)

        pltpu.make_async_copy(sim_hbm.at[base + r],
                              rows_v.at[pl.ds(par, N)], sem).wait()

        for i in range(16):
            hist_v[pl.ds(i * 16, 16)] = zeros16
        for ch in range(NCH):
            x = rows_v[pl.ds(par + ch * 16, 16)]
            ub = lax.bitcast_convert_type(x, jnp.uint32)
            neg = x < 0.0
            u = jnp.where(neg, ~ub, ub | jnp.uint32(0x80000000))
            u_v[pl.ds(ch * 16, 16)] = u
            byte = (u >> jnp.uint32(24)).astype(jnp.int32)
            plsc.addupdate_scatter(hist_v, [byte], ones)

        bbin, k_rem = find_bin(jnp.int32(K))
        prefix = bbin.astype(jnp.uint32) << jnp.uint32(24)

        b0 = bbin.astype(jnp.uint32)
        def comp_body(ch, offv):
            u = u_v[pl.ds(ch * 16, 16)]
            m = (u >> jnp.uint32(24)) == b0
            mi = m.astype(jnp.int32)
            tgt = jnp.maximum(offv + plsc.cumsum(mi) - 1, 0)
            plsc.store_scatter(cand_v, [tgt],
                               lax.bitcast_convert_type(u, jnp.int32), mask=m)
            return offv + plsc.all_reduce_population_count(m)
        cnt_v = lax.fori_loop(0, NCH, comp_body, jnp.zeros((16,), jnp.int32))
        cnt = jnp.max(cnt_v)
        ncc = (cnt + 15) // 16

        for lvl in range(1, 4):
            shift = jnp.uint32(24 - 8 * lvl)
            hi_shift = jnp.uint32(32 - 8 * lvl)
            pref_hi = prefix >> hi_shift
            for i in range(16):
                hist_v[pl.ds(i * 16, 16)] = zeros16

            def ch_body(ch, _, shift=shift, hi_shift=hi_shift,
                        pref_hi=pref_hi):
                u = lax.bitcast_convert_type(cand_v[pl.ds(ch * 16, 16)],
                                             jnp.uint32)
                inb = (ch * 16 + lanes) < cnt
                active = jnp.logical_and(inb, (u >> hi_shift) == pref_hi)
                byte = ((u >> shift) & jnp.uint32(0xFF)).astype(jnp.int32)
                plsc.addupdate_scatter(hist_v, [byte], ones, mask=active)
                return 0

            lax.fori_loop(0, ncc, ch_body, 0)
            bbin, k_rem = find_bin(k_rem)
            prefix = prefix | (bbin.astype(jnp.uint32) << shift)

        pending = jnp.where(lanes == (r % 16), prefix, pending)

        @pl.when(r % 16 == 15)
        def _():
            thru_v[pl.ds((r // 16) * 16, 16)] = pending

        return pending

    lax.fori_loop(0, rpw, row_body, jnp.zeros((16,), jnp.uint32))

    for ch in range(rpw // 16):
        u = thru_v[pl.ds(ch * 16, 16)]
        pos_f = (u >> jnp.uint32(31)) > jnp.uint32(0)
        bits = jnp.where(pos_f, u & jnp.uint32(0x7FFFFFFF), ~u)
        thrf_v[pl.ds(ch * 16, 16)] = lax.bitcast_convert_type(bits, jnp.float32)
    pltpu.sync_copy(thrf_v, thr_hbm.at[pl.ds(base, rpw)])


def _make_kth_sc(rows):
    rpw = rows // NW

    @functools.partial(
        pl.kernel,
        mesh=plsc.VectorSubcoreMesh(core_axis_name="c", subcore_axis_name="s"),
        compiler_params=pltpu.CompilerParams(needs_layout_passes=False),
        out_type=jax.ShapeDtypeStruct((rows,), jnp.float32),
        scratch_types=[
            pltpu.VMEM((2 * N,), jnp.float32),
            pltpu.VMEM((N,), jnp.uint32),
            pltpu.VMEM((N + 16,), jnp.int32),
            pltpu.VMEM((256,), jnp.int32),
            pltpu.VMEM((rpw,), jnp.uint32),
            pltpu.VMEM((rpw,), jnp.float32),
            pltpu.SemaphoreType.DMA,
        ],
    )
    def k(sim_hbm, thr_hbm, rows_v, u_v, cand_v, hist_v, thru_v, thrf_v, sem):
        _kth_sc_body(sim_hbm, thr_hbm, rows_v, u_v, cand_v, hist_v, thru_v,
                     thrf_v, sem, rpw)

    return k


NSPLIT = 2
BH = B // NSPLIT
_kth_sc_half = _make_kth_sc(BH * N)


@jax.jit
def kernel(x, W, bias):
    b, c, h, w = x.shape
    n = h * w
    xr = x.reshape(b, c, n)
    brow = bias.reshape(2 * c, 1)

    feat, pos = pl.pallas_call(
        _featpos_body,
        grid=(b, n // NT_PROJ),
        in_specs=[
            pl.BlockSpec((1, c, NT_PROJ), lambda i, j: (i, 0, j)),
            pl.BlockSpec((2 * c, c), lambda i, j: (0, 0)),
            pl.BlockSpec((2 * c, 1), lambda i, j: (0, 0)),
        ],
        out_specs=[
            pl.BlockSpec((1, c, NT_PROJ), lambda i, j: (i, 0, j)),
            pl.BlockSpec((1, c, NT_PROJ), lambda i, j: (i, 0, j)),
        ],
        out_shape=[
            jax.ShapeDtypeStruct((b, c, n), jnp.float32),
            jax.ShapeDtypeStruct((b, c, n), jnp.float32),
        ],
    )(xr, W, brow)

    outs = []
    for hh in range(NSPLIT):
        pos_h = pos[hh * BH:(hh + 1) * BH]
        feat_h = feat[hh * BH:(hh + 1) * BH]
        sim_h = pl.pallas_call(
            _sim_body,
            grid=(BH, n // T_AGG),
            in_specs=[
                pl.BlockSpec((1, c, T_AGG), lambda i, j: (i, 0, j)),
                pl.BlockSpec((1, c, n), lambda i, j: (i, 0, 0)),
            ],
            out_specs=pl.BlockSpec((1, T_AGG, n), lambda i, j: (i, j, 0)),
            out_shape=jax.ShapeDtypeStruct((BH, n, n), jnp.float32),
        )(pos_h, pos_h)

        thr_h = _kth_sc_half(sim_h.reshape(BH * N, N))
        thr3_h = thr_h.reshape(BH * n // T_AGG, 1, T_AGG)

        out_h = pl.pallas_call(
            _agg_body,
            grid=(BH, n // T_AGG),
            in_specs=[
                pl.BlockSpec((1, T_AGG, n), lambda i, j: (i, j, 0)),
                pl.BlockSpec((1, 1, T_AGG),
                             lambda i, j: (i * (N // T_AGG) + j, 0, 0)),
                pl.BlockSpec((1, c, n), lambda i, j: (i, 0, 0)),
            ],
            out_specs=pl.BlockSpec((1, c, T_AGG), lambda i, j: (i, 0, j)),
            out_shape=jax.ShapeDtypeStruct((BH, c, n), jnp.float32),
        )(sim_h, thr3_h, feat_h)
        outs.append(out_h)

    out = jnp.concatenate(outs, axis=0)
    return out.reshape(b, c, h, w)


# speedup vs baseline: 1.0726x; 1.0726x over previous
import functools
import jax
import jax.numpy as jnp
from jax import lax
from jax.experimental import pallas as pl
from jax.experimental.pallas import tpu as pltpu
from jax.experimental.pallas import tpu_sc as plsc

C = 768
N = 1024
K = 32
B = 8
NT_PROJ = 256
T_AGG = 128

NW = 32
ROWS = B * N
RPW = ROWS // NW
NCH = N // 16


def _featpos_body(x_ref, w_ref, b_ref, feat_ref, pos_ref):
    xb = x_ref[0]
    w = w_ref[...]
    fp = lax.dot_general(w, xb, (((1,), (0,)), ((), ())),
                         preferred_element_type=jnp.float32)
    fp = fp + b_ref[...]
    feat = fp[:C, :]
    posu = fp[C:, :]
    ss = jnp.sum(posu * posu, axis=0, keepdims=True)
    inv = 1.0 / jnp.clip(jnp.sqrt(ss), 1e-12)
    feat_ref[0] = feat
    pos_ref[0] = posu * inv


def _sim_body(pos_t_ref, pos_ref, sim_ref):
    sim_ref[0] = lax.dot_general(pos_t_ref[0], pos_ref[0],
                                 (((0,), (0,)), ((), ())),
                                 preferred_element_type=jnp.float32)


def _agg_body(sim_ref, thr_ref, feat_ref, out_ref):
    sim = sim_ref[0]
    thr = thr_ref[0, 0]
    mask = sim >= thr[:, None]
    e = jnp.where(mask, jnp.exp(sim - 1.0), 0.0)
    s = jnp.sum(e, axis=1, keepdims=True)
    attn = e / s
    out_ref[0] = lax.dot_general(feat_ref[0], attn, (((1,), (1,)), ((), ())),
                                 preferred_element_type=jnp.float32)


def _kth_sc_body(sim_hbm, thr_hbm, rows_v, u_v, cand_v, hist_v, thru_v,
                 thrf_v, sem, rpw):
    wid = lax.axis_index("s") * 2 + lax.axis_index("c")
    base = wid * rpw
    lanes = lax.iota(jnp.int32, 16)
    ones = jnp.ones((16,), jnp.int32)
    zeros16 = jnp.zeros((16,), jnp.int32)

    def find_bin(k_rem):
        tv = jnp.zeros((16,), jnp.int32)
        for j in range(16):
            tv = jnp.where(lanes == j, jnp.sum(hist_v[pl.ds(j * 16, 16)]), tv)
        rev_tv = lax.rev(tv, (0,))
        cst = plsc.cumsum(rev_tv)
        lane_rev = jnp.max(plsc.all_reduce_ffs(cst >= k_rem))
        jstar = 15 - lane_rev
        accstar = jnp.sum(jnp.where(lanes == lane_rev, cst - rev_tv, 0))
        h = hist_v[pl.ds(jstar * 16, 16)]
        rev = lax.rev(h, (0,))
        cs = plsc.cumsum(rev)
        lane2 = jnp.max(plsc.all_reduce_ffs((accstar + cs) >= k_rem))
        cnt_gt = accstar + jnp.sum(jnp.where(lanes == lane2, cs - rev, 0))
        bbin = jstar * 16 + (15 - lane2)
        return bbin, k_rem - cnt_gt

    pltpu.async_copy(sim_hbm.at[base], rows_v.at[pl.ds(0, N)], sem)

    def row_body(r, pending):
        par = (r % 2) * N

        @pl.when(r + 1 < rpw)
        def _():
            nxt = ((r + 1) % 2) * N
            pltpu.async_copy(sim_hbm.at[base + r + 1],
                             rows_v.at[pl.ds(nxt, N)], sem)

        pltpu.make_async_copy(sim_hbm.at[base + r],
                              rows_v.at[pl.ds(par, N)], sem).wait()

        for i in range(16):
            hist_v[pl.ds(i * 16, 16)] = zeros16
        for ch in range(NCH):
            x = rows_v[pl.ds(par + ch * 16, 16)]
            ub = lax.bitcast_convert_type(x, jnp.uint32)
            neg = x < 0.0
            u = jnp.where(neg, ~ub, ub | jnp.uint32(0x80000000))
            u_v[pl.ds(ch * 16, 16)] = u
            byte = (u >> jnp.uint32(24)).astype(jnp.int32)
            plsc.addupdate_scatter(hist_v, [byte], ones)

        bbin, k_rem = find_bin(jnp.int32(K))
        prefix = bbin.astype(jnp.uint32) << jnp.uint32(24)

        b0 = bbin.astype(jnp.uint32)
        def comp_body(ch, off):
            u = u_v[pl.ds(ch * 16, 16)]
            m = (u >> jnp.uint32(24)) == b0
            plsc.store_compressed(
                cand_v.at[pl.ds(off, 16)],
                lax.bitcast_convert_type(u, jnp.int32), mask=m)
            return off + plsc.all_reduce_population_count(m)[0]
        cnt = lax.fori_loop(0, NCH, comp_body, jnp.int32(0))
        ncc = (cnt + 15) // 16

        for lvl in range(1, 4):
            shift = jnp.uint32(24 - 8 * lvl)
            hi_shift = jnp.uint32(32 - 8 * lvl)
            pref_hi = prefix >> hi_shift
            for i in range(16):
                hist_v[pl.ds(i * 16, 16)] = zeros16

            def ch_body(ch, _, shift=shift, hi_shift=hi_shift,
                        pref_hi=pref_hi):
                u = lax.bitcast_convert_type(cand_v[pl.ds(ch * 16, 16)],
                                             jnp.uint32)
                inb = (ch * 16 + lanes) < cnt
                active = jnp.logical_and(inb, (u >> hi_shift) == pref_hi)
                byte = ((u >> shift) & jnp.uint32(0xFF)).astype(jnp.int32)
                plsc.addupdate_scatter(hist_v, [byte], ones, mask=active)
                return 0

            lax.fori_loop(0, ncc, ch_body, 0)
            bbin, k_rem = find_bin(k_rem)
            prefix = prefix | (bbin.astype(jnp.uint32) << shift)

        pending = jnp.where(lanes == (r % 16), prefix, pending)

        @pl.when(r % 16 == 15)
        def _():
            thru_v[pl.ds((r // 16) * 16, 16)] = pending

        return pending

    lax.fori_loop(0, rpw, row_body, jnp.zeros((16,), jnp.uint32))

    for ch in range(rpw // 16):
        u = thru_v[pl.ds(ch * 16, 16)]
        pos_f = (u >> jnp.uint32(31)) > jnp.uint32(0)
        bits = jnp.where(pos_f, u & jnp.uint32(0x7FFFFFFF), ~u)
        thrf_v[pl.ds(ch * 16, 16)] = lax.bitcast_convert_type(bits, jnp.float32)
    pltpu.sync_copy(thrf_v, thr_hbm.at[pl.ds(base, rpw)])


def _make_kth_sc(rows):
    rpw = rows // NW

    @functools.partial(
        pl.kernel,
        mesh=plsc.VectorSubcoreMesh(core_axis_name="c", subcore_axis_name="s"),
        compiler_params=pltpu.CompilerParams(needs_layout_passes=False),
        out_type=jax.ShapeDtypeStruct((rows,), jnp.float32),
        scratch_types=[
            pltpu.VMEM((2 * N,), jnp.float32),
            pltpu.VMEM((N,), jnp.uint32),
            pltpu.VMEM((N + 16,), jnp.int32),
            pltpu.VMEM((256,), jnp.int32),
            pltpu.VMEM((rpw,), jnp.uint32),
            pltpu.VMEM((rpw,), jnp.float32),
            pltpu.SemaphoreType.DMA,
        ],
    )
    def k(sim_hbm, thr_hbm, rows_v, u_v, cand_v, hist_v, thru_v, thrf_v, sem):
        _kth_sc_body(sim_hbm, thr_hbm, rows_v, u_v, cand_v, hist_v, thru_v,
                     thrf_v, sem, rpw)

    return k


NSPLIT = 2
BH = B // NSPLIT
_kth_sc_half = _make_kth_sc(BH * N)


@jax.jit
def kernel(x, W, bias):
    b, c, h, w = x.shape
    n = h * w
    xr = x.reshape(b, c, n)
    brow = bias.reshape(2 * c, 1)

    feat, pos = pl.pallas_call(
        _featpos_body,
        grid=(b, n // NT_PROJ),
        in_specs=[
            pl.BlockSpec((1, c, NT_PROJ), lambda i, j: (i, 0, j)),
            pl.BlockSpec((2 * c, c), lambda i, j: (0, 0)),
            pl.BlockSpec((2 * c, 1), lambda i, j: (0, 0)),
        ],
        out_specs=[
            pl.BlockSpec((1, c, NT_PROJ), lambda i, j: (i, 0, j)),
            pl.BlockSpec((1, c, NT_PROJ), lambda i, j: (i, 0, j)),
        ],
        out_shape=[
            jax.ShapeDtypeStruct((b, c, n), jnp.float32),
            jax.ShapeDtypeStruct((b, c, n), jnp.float32),
        ],
    )(xr, W, brow)

    outs = []
    for hh in range(NSPLIT):
        pos_h = pos[hh * BH:(hh + 1) * BH]
        feat_h = feat[hh * BH:(hh + 1) * BH]
        sim_h = pl.pallas_call(
            _sim_body,
            grid=(BH, n // T_AGG),
            in_specs=[
                pl.BlockSpec((1, c, T_AGG), lambda i, j: (i, 0, j)),
                pl.BlockSpec((1, c, n), lambda i, j: (i, 0, 0)),
            ],
            out_specs=pl.BlockSpec((1, T_AGG, n), lambda i, j: (i, j, 0)),
            out_shape=jax.ShapeDtypeStruct((BH, n, n), jnp.float32),
        )(pos_h, pos_h)

        thr_h = _kth_sc_half(sim_h.reshape(BH * N, N))
        thr3_h = thr_h.reshape(BH * n // T_AGG, 1, T_AGG)

        out_h = pl.pallas_call(
            _agg_body,
            grid=(BH, n // T_AGG),
            in_specs=[
                pl.BlockSpec((1, T_AGG, n), lambda i, j: (i, j, 0)),
                pl.BlockSpec((1, 1, T_AGG),
                             lambda i, j: (i * (N // T_AGG) + j, 0, 0)),
                pl.BlockSpec((1, c, n), lambda i, j: (i, 0, 0)),
            ],
            out_specs=pl.BlockSpec((1, c, T_AGG), lambda i, j: (i, 0, j)),
            out_shape=jax.ShapeDtypeStruct((BH, c, n), jnp.float32),
        )(sim_h, thr3_h, feat_h)
        outs.append(out_h)

    out = jnp.concatenate(outs, axis=0)
    return out.reshape(b, c, h, w)


# speedup vs baseline: 1.1386x; 1.0616x over previous
import functools
import jax
import jax.numpy as jnp
from jax import lax
from jax.experimental import pallas as pl
from jax.experimental.pallas import tpu as pltpu
from jax.experimental.pallas import tpu_sc as plsc

C = 768
N = 1024
K = 32
B = 8
NT_PROJ = 256
T_AGG = 128

NW = 32
ROWS = B * N
RPW = ROWS // NW
NCH = N // 16


def _featpos_body(x_ref, w_ref, b_ref, feat_ref, pos_ref):
    xb = x_ref[0]
    w = w_ref[...]
    fp = lax.dot_general(w, xb, (((1,), (0,)), ((), ())),
                         preferred_element_type=jnp.float32)
    fp = fp + b_ref[...]
    feat = fp[:C, :]
    posu = fp[C:, :]
    ss = jnp.sum(posu * posu, axis=0, keepdims=True)
    inv = 1.0 / jnp.clip(jnp.sqrt(ss), 1e-12)
    feat_ref[0] = feat
    pos_ref[0] = posu * inv


def _sim_body(pos_t_ref, pos_ref, sim_ref):
    sim_ref[0] = lax.dot_general(pos_t_ref[0], pos_ref[0],
                                 (((0,), (0,)), ((), ())),
                                 preferred_element_type=jnp.float32)


def _agg_body(sim_ref, thr_ref, feat_ref, out_ref):
    sim = sim_ref[0]
    thr = thr_ref[0, 0]
    mask = sim >= thr[:, None]
    e = jnp.where(mask, jnp.exp(sim - 1.0), 0.0)
    s = jnp.sum(e, axis=1, keepdims=True)
    attn = e / s
    out_ref[0] = lax.dot_general(feat_ref[0], attn, (((1,), (1,)), ((), ())),
                                 preferred_element_type=jnp.float32)


def _kth_sc_body(sim_hbm, thr_hbm, rows_v, u_v, cand_v, hist_v, thru_v,
                 thrf_v, sem, rpw):
    wid = lax.axis_index("s") * 2 + lax.axis_index("c")
    base = wid * rpw
    lanes = lax.iota(jnp.int32, 16)
    ones = jnp.ones((16,), jnp.int32)
    zeros16 = jnp.zeros((16,), jnp.int32)

    def find_bin(k_rem):
        tv = jnp.zeros((16,), jnp.int32)
        for j in range(16):
            tv = jnp.where(lanes == j, jnp.sum(hist_v[pl.ds(j * 16, 16)]), tv)
        rev_tv = lax.rev(tv, (0,))
        cst = plsc.cumsum(rev_tv)
        lane_rev = jnp.max(plsc.all_reduce_ffs(cst >= k_rem))
        jstar = 15 - lane_rev
        accstar = jnp.sum(jnp.where(lanes == lane_rev, cst - rev_tv, 0))
        h = hist_v[pl.ds(jstar * 16, 16)]
        rev = lax.rev(h, (0,))
        cs = plsc.cumsum(rev)
        lane2 = jnp.max(plsc.all_reduce_ffs((accstar + cs) >= k_rem))
        cnt_gt = accstar + jnp.sum(jnp.where(lanes == lane2, cs - rev, 0))
        bbin = jstar * 16 + (15 - lane2)
        return bbin, k_rem - cnt_gt

    pltpu.async_copy(sim_hbm.at[base], rows_v.at[pl.ds(0, N)], sem)

    def row_body(r, pending):
        par = (r % 2) * N

        @pl.when(r + 1 < rpw)
        def _():
            nxt = ((r + 1) % 2) * N
            pltpu.async_copy(sim_hbm.at[base + r + 1],
                             rows_v.at[pl.ds(nxt, N)], sem)

        pltpu.make_async_copy(sim_hbm.at[base + r],
                              rows_v.at[pl.ds(par, N)], sem).wait()

        for i in range(16):
            hist_v[pl.ds(i * 16, 16)] = zeros16
        for ch in range(NCH):
            x = rows_v[pl.ds(par + ch * 16, 16)]
            ub = lax.bitcast_convert_type(x, jnp.uint32)
            neg = x < 0.0
            u = jnp.where(neg, ~ub, ub | jnp.uint32(0x80000000))
            u_v[pl.ds(ch * 16, 16)] = u
            byte = (u >> jnp.uint32(24)).astype(jnp.int32)
            plsc.addupdate_scatter(hist_v, [byte], ones)

        bbin, k_rem = find_bin(jnp.int32(K))
        prefix = bbin.astype(jnp.uint32) << jnp.uint32(24)

        b0 = bbin.astype(jnp.uint32)
        def comp_body(ch, off):
            u = u_v[pl.ds(ch * 16, 16)]
            m = (u >> jnp.uint32(24)) == b0
            plsc.store_compressed(cand_v.at[pl.ds(off, 16)], u, mask=m)
            return off + jnp.sum(m.astype(jnp.int32))
        cnt = lax.fori_loop(0, NCH, comp_body, jnp.int32(0))
        ncc = (cnt + 15) // 16

        for lvl in range(1, 4):
            shift = jnp.uint32(24 - 8 * lvl)
            hi_shift = jnp.uint32(32 - 8 * lvl)
            pref_hi = prefix >> hi_shift
            for i in range(16):
                hist_v[pl.ds(i * 16, 16)] = zeros16

            def ch_body(ch, _, shift=shift, hi_shift=hi_shift,
                        pref_hi=pref_hi):
                u = cand_v[pl.ds(ch * 16, 16)]
                inb = (ch * 16 + lanes) < cnt
                active = jnp.logical_and(inb, (u >> hi_shift) == pref_hi)
                byte = ((u >> shift) & jnp.uint32(0xFF)).astype(jnp.int32)
                plsc.addupdate_scatter(hist_v, [byte], ones, mask=active)
                return 0

            lax.fori_loop(0, ncc, ch_body, 0)
            bbin, k_rem = find_bin(k_rem)
            prefix = prefix | (bbin.astype(jnp.uint32) << shift)

        pending = jnp.where(lanes == (r % 16), prefix, pending)

        @pl.when(r % 16 == 15)
        def _():
            thru_v[pl.ds((r // 16) * 16, 16)] = pending

        return pending

    lax.fori_loop(0, rpw, row_body, jnp.zeros((16,), jnp.uint32))

    for ch in range(rpw // 16):
        u = thru_v[pl.ds(ch * 16, 16)]
        pos_f = (u >> jnp.uint32(31)) > jnp.uint32(0)
        bits = jnp.where(pos_f, u & jnp.uint32(0x7FFFFFFF), ~u)
        thrf_v[pl.ds(ch * 16, 16)] = lax.bitcast_convert_type(bits, jnp.float32)
    pltpu.sync_copy(thrf_v, thr_hbm.at[pl.ds(base, rpw)])


def _make_kth_sc(rows):
    rpw = rows // NW

    @functools.partial(
        pl.kernel,
        mesh=plsc.VectorSubcoreMesh(core_axis_name="c", subcore_axis_name="s"),
        compiler_params=pltpu.CompilerParams(needs_layout_passes=False),
        out_type=jax.ShapeDtypeStruct((rows,), jnp.float32),
        scratch_types=[
            pltpu.VMEM((2 * N,), jnp.float32),
            pltpu.VMEM((N,), jnp.uint32),
            pltpu.VMEM((N + 16,), jnp.uint32),
            pltpu.VMEM((256,), jnp.int32),
            pltpu.VMEM((rpw,), jnp.uint32),
            pltpu.VMEM((rpw,), jnp.float32),
            pltpu.SemaphoreType.DMA,
        ],
    )
    def k(sim_hbm, thr_hbm, rows_v, u_v, cand_v, hist_v, thru_v, thrf_v, sem):
        _kth_sc_body(sim_hbm, thr_hbm, rows_v, u_v, cand_v, hist_v, thru_v,
                     thrf_v, sem, rpw)

    return k


NSPLIT = 4
BH = B // NSPLIT
_kth_sc_half = _make_kth_sc(BH * N)


@jax.jit
def kernel(x, W, bias):
    b, c, h, w = x.shape
    n = h * w
    xr = x.reshape(b, c, n)
    brow = bias.reshape(2 * c, 1)

    feat, pos = pl.pallas_call(
        _featpos_body,
        grid=(b, n // NT_PROJ),
        in_specs=[
            pl.BlockSpec((1, c, NT_PROJ), lambda i, j: (i, 0, j)),
            pl.BlockSpec((2 * c, c), lambda i, j: (0, 0)),
            pl.BlockSpec((2 * c, 1), lambda i, j: (0, 0)),
        ],
        out_specs=[
            pl.BlockSpec((1, c, NT_PROJ), lambda i, j: (i, 0, j)),
            pl.BlockSpec((1, c, NT_PROJ), lambda i, j: (i, 0, j)),
        ],
        out_shape=[
            jax.ShapeDtypeStruct((b, c, n), jnp.float32),
            jax.ShapeDtypeStruct((b, c, n), jnp.float32),
        ],
    )(xr, W, brow)

    outs = []
    for hh in range(NSPLIT):
        pos_h = pos[hh * BH:(hh + 1) * BH]
        feat_h = feat[hh * BH:(hh + 1) * BH]
        sim_h = pl.pallas_call(
            _sim_body,
            grid=(BH, n // T_AGG),
            in_specs=[
                pl.BlockSpec((1, c, T_AGG), lambda i, j: (i, 0, j)),
                pl.BlockSpec((1, c, n), lambda i, j: (i, 0, 0)),
            ],
            out_specs=pl.BlockSpec((1, T_AGG, n), lambda i, j: (i, j, 0)),
            out_shape=jax.ShapeDtypeStruct((BH, n, n), jnp.float32),
        )(pos_h, pos_h)

        thr_h = _kth_sc_half(sim_h.reshape(BH * N, N))
        thr3_h = thr_h.reshape(BH * n // T_AGG, 1, T_AGG)

        out_h = pl.pallas_call(
            _agg_body,
            grid=(BH, n // T_AGG),
            in_specs=[
                pl.BlockSpec((1, T_AGG, n), lambda i, j: (i, j, 0)),
                pl.BlockSpec((1, 1, T_AGG),
                             lambda i, j: (i * (N // T_AGG) + j, 0, 0)),
                pl.BlockSpec((1, c, n), lambda i, j: (i, 0, 0)),
            ],
            out_specs=pl.BlockSpec((1, c, T_AGG), lambda i, j: (i, 0, j)),
            out_shape=jax.ShapeDtypeStruct((BH, c, n), jnp.float32),
        )(sim_h, thr3_h, feat_h)
        outs.append(out_h)

    out = jnp.concatenate(outs, axis=0)
    return out.reshape(b, c, h, w)


# speedup vs baseline: 1.1464x; 1.0068x over previous
import functools
import jax
import jax.numpy as jnp
from jax import lax
from jax.experimental import pallas as pl
from jax.experimental.pallas import tpu as pltpu
from jax.experimental.pallas import tpu_sc as plsc

C = 768
N = 1024
K = 32
B = 8
NT_PROJ = 256
T_AGG = 128

NW = 32
ROWS = B * N
RPW = ROWS // NW
NCH = N // 16


def _featpos_body(x_ref, w_ref, b_ref, feat_ref, pos_ref):
    xb = x_ref[0]
    w = w_ref[...]
    fp = lax.dot_general(w, xb, (((1,), (0,)), ((), ())),
                         preferred_element_type=jnp.float32)
    fp = fp + b_ref[...]
    feat = fp[:C, :]
    posu = fp[C:, :]
    ss = jnp.sum(posu * posu, axis=0, keepdims=True)
    inv = 1.0 / jnp.clip(jnp.sqrt(ss), 1e-12)
    feat_ref[0] = feat
    pos_ref[0] = posu * inv


def _sim_body(pos_t_ref, pos_ref, sim_ref):
    sim_ref[0] = lax.dot_general(pos_t_ref[0], pos_ref[0],
                                 (((0,), (0,)), ((), ())),
                                 preferred_element_type=jnp.float32)


def _agg_body(sim_ref, thr_ref, feat_ref, out_ref):
    sim = sim_ref[0]
    thr = thr_ref[0, 0]
    mask = sim >= thr[:, None]
    e = jnp.where(mask, jnp.exp(sim - 1.0), 0.0)
    s = jnp.sum(e, axis=1, keepdims=True)
    attn = e / s
    out_ref[0] = lax.dot_general(feat_ref[0], attn, (((1,), (1,)), ((), ())),
                                 preferred_element_type=jnp.float32)


def _kth_sc_body(sim_hbm, thr_hbm, rows_v, u_v, cand_v, hist_v, thru_v,
                 thrf_v, sem, rpw):
    wid = lax.axis_index("s") * 2 + lax.axis_index("c")
    base = wid * rpw
    lanes = lax.iota(jnp.int32, 16)
    ones = jnp.ones((16,), jnp.int32)
    zeros16 = jnp.zeros((16,), jnp.int32)

    def find_bin(k_rem):
        tv = jnp.zeros((16,), jnp.int32)
        for j in range(16):
            tv = jnp.where(lanes == j, jnp.sum(hist_v[pl.ds(j * 16, 16)]), tv)
        rev_tv = lax.rev(tv, (0,))
        cst = plsc.cumsum(rev_tv)
        lane_rev = jnp.max(plsc.all_reduce_ffs(cst >= k_rem))
        jstar = 15 - lane_rev
        accstar = jnp.sum(jnp.where(lanes == lane_rev, cst - rev_tv, 0))
        h = hist_v[pl.ds(jstar * 16, 16)]
        rev = lax.rev(h, (0,))
        cs = plsc.cumsum(rev)
        lane2 = jnp.max(plsc.all_reduce_ffs((accstar + cs) >= k_rem))
        cnt_gt = accstar + jnp.sum(jnp.where(lanes == lane2, cs - rev, 0))
        bbin = jstar * 16 + (15 - lane2)
        return bbin, k_rem - cnt_gt

    pltpu.async_copy(sim_hbm.at[base], rows_v.at[pl.ds(0, N)], sem)

    def row_body(r, pending):
        par = (r % 2) * N

        @pl.when(r + 1 < rpw)
        def _():
            nxt = ((r + 1) % 2) * N
            pltpu.async_copy(sim_hbm.at[base + r + 1],
                             rows_v.at[pl.ds(nxt, N)], sem)

        pltpu.make_async_copy(sim_hbm.at[base + r],
                              rows_v.at[pl.ds(par, N)], sem).wait()

        for i in range(16):
            hist_v[pl.ds(i * 16, 16)] = zeros16
        for ch in range(NCH):
            x = rows_v[pl.ds(par + ch * 16, 16)]
            ub = lax.bitcast_convert_type(x, jnp.uint32)
            neg = x < 0.0
            u = jnp.where(neg, ~ub, ub | jnp.uint32(0x80000000))
            u_v[pl.ds(ch * 16, 16)] = u
            byte = (u >> jnp.uint32(24)).astype(jnp.int32)
            plsc.addupdate_scatter(hist_v, [byte], ones)

        bbin, k_rem = find_bin(jnp.int32(K))
        prefix = bbin.astype(jnp.uint32) << jnp.uint32(24)

        b0 = bbin.astype(jnp.uint32)
        def comp_body(ch, off):
            u = u_v[pl.ds(ch * 16, 16)]
            m = (u >> jnp.uint32(24)) == b0
            plsc.store_compressed(cand_v.at[pl.ds(off, 16)], u, mask=m)
            return off + jnp.sum(m.astype(jnp.int32))
        cnt = lax.fori_loop(0, NCH, comp_body, jnp.int32(0))
        ncc = (cnt + 15) // 16

        for lvl in range(1, 4):
            shift = jnp.uint32(24 - 8 * lvl)
            hi_shift = jnp.uint32(32 - 8 * lvl)
            pref_hi = prefix >> hi_shift
            for i in range(16):
                hist_v[pl.ds(i * 16, 16)] = zeros16

            def ch_body(ch, _, shift=shift, hi_shift=hi_shift,
                        pref_hi=pref_hi):
                u = cand_v[pl.ds(ch * 16, 16)]
                inb = (ch * 16 + lanes) < cnt
                active = jnp.logical_and(inb, (u >> hi_shift) == pref_hi)
                byte = ((u >> shift) & jnp.uint32(0xFF)).astype(jnp.int32)
                plsc.addupdate_scatter(hist_v, [byte], ones, mask=active)
                return 0

            lax.fori_loop(0, ncc, ch_body, 0)
            bbin, k_rem = find_bin(k_rem)
            prefix = prefix | (bbin.astype(jnp.uint32) << shift)

        pending = jnp.where(lanes == (r % 16), prefix, pending)

        @pl.when(r % 16 == 15)
        def _():
            thru_v[pl.ds((r // 16) * 16, 16)] = pending

        return pending

    lax.fori_loop(0, rpw, row_body, jnp.zeros((16,), jnp.uint32))

    for ch in range(rpw // 16):
        u = thru_v[pl.ds(ch * 16, 16)]
        pos_f = (u >> jnp.uint32(31)) > jnp.uint32(0)
        bits = jnp.where(pos_f, u & jnp.uint32(0x7FFFFFFF), ~u)
        thrf_v[pl.ds(ch * 16, 16)] = lax.bitcast_convert_type(bits, jnp.float32)
    pltpu.sync_copy(thrf_v, thr_hbm.at[pl.ds(base, rpw)])


def _make_kth_sc(rows):
    rpw = rows // NW

    @functools.partial(
        pl.kernel,
        mesh=plsc.VectorSubcoreMesh(core_axis_name="c", subcore_axis_name="s"),
        compiler_params=pltpu.CompilerParams(needs_layout_passes=False),
        out_type=jax.ShapeDtypeStruct((rows,), jnp.float32),
        scratch_types=[
            pltpu.VMEM((2 * N,), jnp.float32),
            pltpu.VMEM((N,), jnp.uint32),
            pltpu.VMEM((N + 16,), jnp.uint32),
            pltpu.VMEM((256,), jnp.int32),
            pltpu.VMEM((rpw,), jnp.uint32),
            pltpu.VMEM((rpw,), jnp.float32),
            pltpu.SemaphoreType.DMA,
        ],
    )
    def k(sim_hbm, thr_hbm, rows_v, u_v, cand_v, hist_v, thru_v, thrf_v, sem):
        _kth_sc_body(sim_hbm, thr_hbm, rows_v, u_v, cand_v, hist_v, thru_v,
                     thrf_v, sem, rpw)

    return k


NSPLIT = 8
BH = B // NSPLIT
_kth_sc_half = _make_kth_sc(BH * N)


@jax.jit
def kernel(x, W, bias):
    b, c, h, w = x.shape
    n = h * w
    xr = x.reshape(b, c, n)
    brow = bias.reshape(2 * c, 1)

    feat, pos = pl.pallas_call(
        _featpos_body,
        grid=(b, n // NT_PROJ),
        in_specs=[
            pl.BlockSpec((1, c, NT_PROJ), lambda i, j: (i, 0, j)),
            pl.BlockSpec((2 * c, c), lambda i, j: (0, 0)),
            pl.BlockSpec((2 * c, 1), lambda i, j: (0, 0)),
        ],
        out_specs=[
            pl.BlockSpec((1, c, NT_PROJ), lambda i, j: (i, 0, j)),
            pl.BlockSpec((1, c, NT_PROJ), lambda i, j: (i, 0, j)),
        ],
        out_shape=[
            jax.ShapeDtypeStruct((b, c, n), jnp.float32),
            jax.ShapeDtypeStruct((b, c, n), jnp.float32),
        ],
    )(xr, W, brow)

    outs = []
    for hh in range(NSPLIT):
        pos_h = pos[hh * BH:(hh + 1) * BH]
        feat_h = feat[hh * BH:(hh + 1) * BH]
        sim_h = pl.pallas_call(
            _sim_body,
            grid=(BH, n // T_AGG),
            in_specs=[
                pl.BlockSpec((1, c, T_AGG), lambda i, j: (i, 0, j)),
                pl.BlockSpec((1, c, n), lambda i, j: (i, 0, 0)),
            ],
            out_specs=pl.BlockSpec((1, T_AGG, n), lambda i, j: (i, j, 0)),
            out_shape=jax.ShapeDtypeStruct((BH, n, n), jnp.float32),
        )(pos_h, pos_h)

        thr_h = _kth_sc_half(sim_h.reshape(BH * N, N))
        thr3_h = thr_h.reshape(BH * n // T_AGG, 1, T_AGG)

        out_h = pl.pallas_call(
            _agg_body,
            grid=(BH, n // T_AGG),
            in_specs=[
                pl.BlockSpec((1, T_AGG, n), lambda i, j: (i, j, 0)),
                pl.BlockSpec((1, 1, T_AGG),
                             lambda i, j: (i * (N // T_AGG) + j, 0, 0)),
                pl.BlockSpec((1, c, n), lambda i, j: (i, 0, 0)),
            ],
            out_specs=pl.BlockSpec((1, c, T_AGG), lambda i, j: (i, 0, j)),
            out_shape=jax.ShapeDtypeStruct((BH, c, n), jnp.float32),
        )(sim_h, thr3_h, feat_h)
        outs.append(out_h)

    out = jnp.concatenate(outs, axis=0)
    return out.reshape(b, c, h, w)


# speedup vs baseline: 1.2830x; 1.1192x over previous
import functools
import jax
import jax.numpy as jnp
from jax import lax
from jax.experimental import pallas as pl
from jax.experimental.pallas import tpu as pltpu
from jax.experimental.pallas import tpu_sc as plsc

C = 768
N = 1024
K = 32
B = 8
NT_PROJ = 256
T_AGG = 128

NW = 32
ROWS = B * N
RPW = ROWS // NW
NCH = N // 16


def _featpos_body(x_ref, w_ref, b_ref, feat_ref, pos_ref):
    xb = x_ref[0]
    w = w_ref[...]
    fp = lax.dot_general(w, xb, (((1,), (0,)), ((), ())),
                         preferred_element_type=jnp.float32)
    fp = fp + b_ref[...]
    feat = fp[:C, :]
    posu = fp[C:, :]
    ss = jnp.sum(posu * posu, axis=0, keepdims=True)
    inv = 1.0 / jnp.clip(jnp.sqrt(ss), 1e-12)
    feat_ref[0] = feat
    pos_ref[0] = posu * inv


def _sim_body(pos_t_ref, pos_ref, sim_ref):
    sim_ref[0] = lax.dot_general(pos_t_ref[0], pos_ref[0],
                                 (((0,), (0,)), ((), ())),
                                 preferred_element_type=jnp.float32)


def _agg_body(sim_ref, thr_ref, feat_ref, out_ref):
    sim = sim_ref[0]
    thr = thr_ref[0, 0]
    mask = sim >= thr[:, None]
    e = jnp.where(mask, jnp.exp(sim - 1.0), 0.0)
    s = jnp.sum(e, axis=1, keepdims=True)
    attn = e / s
    out_ref[0] = lax.dot_general(feat_ref[0], attn, (((1,), (1,)), ((), ())),
                                 preferred_element_type=jnp.float32)


NP = N + 16


def _kth_sc_body(sim_hbm, thr_hbm, rows_v, u_v, cand_v, hist_v, thru_v,
                 thrf_v, sem, rpw):
    wid = lax.axis_index("s") * 2 + lax.axis_index("c")
    base = wid * rpw
    half = rpw // 2
    lanes = lax.iota(jnp.int32, 16)
    ones = jnp.ones((16,), jnp.int32)
    zeros16 = jnp.zeros((16,), jnp.int32)

    def find_bin(hoff, k_rem):
        tv = jnp.zeros((16,), jnp.int32)
        for j in range(16):
            tv = jnp.where(lanes == j,
                           jnp.sum(hist_v[pl.ds(hoff + j * 16, 16)]), tv)
        rev_tv = lax.rev(tv, (0,))
        cst = plsc.cumsum(rev_tv)
        lane_rev = jnp.max(plsc.all_reduce_ffs(cst >= k_rem))
        jstar = 15 - lane_rev
        accstar = jnp.sum(jnp.where(lanes == lane_rev, cst - rev_tv, 0))
        h = hist_v[pl.ds(hoff + jstar * 16, 16)]
        rev = lax.rev(h, (0,))
        cs = plsc.cumsum(rev)
        lane2 = jnp.max(plsc.all_reduce_ffs((accstar + cs) >= k_rem))
        cnt_gt = accstar + jnp.sum(jnp.where(lanes == lane2, cs - rev, 0))
        bbin = jstar * 16 + (15 - lane2)
        return bbin, k_rem - cnt_gt

    def conv16(x):
        ub = lax.bitcast_convert_type(x, jnp.uint32)
        return jnp.where(x < 0.0, ~ub, ub | jnp.uint32(0x80000000))

    pltpu.async_copy(sim_hbm.at[base], rows_v.at[pl.ds(0, N)], sem)
    pltpu.async_copy(sim_hbm.at[base + half], rows_v.at[pl.ds(2 * N, N)], sem)

    def row_body(r, carry):
        pend_a, pend_b = carry
        par_a = (r % 2) * N
        par_b = (2 + (r % 2)) * N

        @pl.when(r + 1 < half)
        def _():
            nxt = (r + 1) % 2
            pltpu.async_copy(sim_hbm.at[base + r + 1],
                             rows_v.at[pl.ds(nxt * N, N)], sem)
            pltpu.async_copy(sim_hbm.at[base + half + r + 1],
                             rows_v.at[pl.ds((2 + nxt) * N, N)], sem)

        pltpu.make_async_copy(sim_hbm.at[base + r],
                              rows_v.at[pl.ds(par_a, N)], sem).wait()
        pltpu.make_async_copy(sim_hbm.at[base + half + r],
                              rows_v.at[pl.ds(par_b, N)], sem).wait()

        for i in range(16):
            hist_v[pl.ds(i * 16, 16)] = zeros16
            hist_v[pl.ds(256 + i * 16, 16)] = zeros16
        for ch in range(NCH):
            ua = conv16(rows_v[pl.ds(par_a + ch * 16, 16)])
            ub_ = conv16(rows_v[pl.ds(par_b + ch * 16, 16)])
            u_v[pl.ds(ch * 16, 16)] = ua
            u_v[pl.ds(N + ch * 16, 16)] = ub_
            ba = (ua >> jnp.uint32(24)).astype(jnp.int32)
            bb = (ub_ >> jnp.uint32(24)).astype(jnp.int32)
            plsc.addupdate_scatter(hist_v, [ba], ones)
            plsc.addupdate_scatter(hist_v, [bb + 256], ones)

        bbin_a, kr_a = find_bin(0, jnp.int32(K))
        bbin_b, kr_b = find_bin(256, jnp.int32(K))
        pref_a = bbin_a.astype(jnp.uint32) << jnp.uint32(24)
        pref_b = bbin_b.astype(jnp.uint32) << jnp.uint32(24)

        b0_a = bbin_a.astype(jnp.uint32)
        b0_b = bbin_b.astype(jnp.uint32)

        def comp_body(ch, c):
            off_a, off_b = c
            ua = u_v[pl.ds(ch * 16, 16)]
            ub_ = u_v[pl.ds(N + ch * 16, 16)]
            ma = (ua >> jnp.uint32(24)) == b0_a
            mb = (ub_ >> jnp.uint32(24)) == b0_b
            plsc.store_compressed(cand_v.at[pl.ds(off_a, 16)], ua, mask=ma)
            plsc.store_compressed(cand_v.at[pl.ds(NP + off_b, 16)], ub_,
                                  mask=mb)
            return (off_a + jnp.sum(ma.astype(jnp.int32)),
                    off_b + jnp.sum(mb.astype(jnp.int32)))

        cnt_a, cnt_b = lax.fori_loop(0, NCH, comp_body,
                                     (jnp.int32(0), jnp.int32(0)))
        ncc = (jnp.maximum(cnt_a, cnt_b) + 15) // 16

        for lvl in range(1, 4):
            shift = jnp.uint32(24 - 8 * lvl)
            hi_shift = jnp.uint32(32 - 8 * lvl)
            ph_a = pref_a >> hi_shift
            ph_b = pref_b >> hi_shift
            for i in range(16):
                hist_v[pl.ds(i * 16, 16)] = zeros16
                hist_v[pl.ds(256 + i * 16, 16)] = zeros16

            def ch_body(ch, _, shift=shift, hi_shift=hi_shift,
                        ph_a=ph_a, ph_b=ph_b):
                ua = cand_v[pl.ds(ch * 16, 16)]
                ub_ = cand_v[pl.ds(NP + ch * 16, 16)]
                inb = ch * 16 + lanes
                act_a = jnp.logical_and(inb < cnt_a,
                                        (ua >> hi_shift) == ph_a)
                act_b = jnp.logical_and(inb < cnt_b,
                                        (ub_ >> hi_shift) == ph_b)
                bya = ((ua >> shift) & jnp.uint32(0xFF)).astype(jnp.int32)
                byb = ((ub_ >> shift) & jnp.uint32(0xFF)).astype(jnp.int32)
                plsc.addupdate_scatter(hist_v, [bya], ones, mask=act_a)
                plsc.addupdate_scatter(hist_v, [byb + 256], ones, mask=act_b)
                return 0

            lax.fori_loop(0, ncc, ch_body, 0)
            bbin_a, kr_a = find_bin(0, kr_a)
            bbin_b, kr_b = find_bin(256, kr_b)
            pref_a = pref_a | (bbin_a.astype(jnp.uint32) << shift)
            pref_b = pref_b | (bbin_b.astype(jnp.uint32) << shift)

        pend_a = jnp.where(lanes == (r % 16), pref_a, pend_a)
        pend_b = jnp.where(lanes == (r % 16), pref_b, pend_b)

        @pl.when(r % 16 == 15)
        def _():
            thru_v[pl.ds((r // 16) * 16, 16)] = pend_a
            thru_v[pl.ds(half + (r // 16) * 16, 16)] = pend_b

        return (pend_a, pend_b)

    z16 = jnp.zeros((16,), jnp.uint32)
    lax.fori_loop(0, half, row_body, (z16, z16))

    for ch in range(rpw // 16):
        u = thru_v[pl.ds(ch * 16, 16)]
        pos_f = (u >> jnp.uint32(31)) > jnp.uint32(0)
        bits = jnp.where(pos_f, u & jnp.uint32(0x7FFFFFFF), ~u)
        thrf_v[pl.ds(ch * 16, 16)] = lax.bitcast_convert_type(bits, jnp.float32)
    pltpu.sync_copy(thrf_v, thr_hbm.at[pl.ds(base, rpw)])


def _make_kth_sc(rows):
    rpw = rows // NW

    @functools.partial(
        pl.kernel,
        mesh=plsc.VectorSubcoreMesh(core_axis_name="c", subcore_axis_name="s"),
        compiler_params=pltpu.CompilerParams(needs_layout_passes=False),
        out_type=jax.ShapeDtypeStruct((rows,), jnp.float32),
        scratch_types=[
            pltpu.VMEM((4 * N,), jnp.float32),
            pltpu.VMEM((2 * N,), jnp.uint32),
            pltpu.VMEM((2 * NP,), jnp.uint32),
            pltpu.VMEM((512,), jnp.int32),
            pltpu.VMEM((rpw,), jnp.uint32),
            pltpu.VMEM((rpw,), jnp.float32),
            pltpu.SemaphoreType.DMA,
        ],
    )
    def k(sim_hbm, thr_hbm, rows_v, u_v, cand_v, hist_v, thru_v, thrf_v, sem):
        _kth_sc_body(sim_hbm, thr_hbm, rows_v, u_v, cand_v, hist_v, thru_v,
                     thrf_v, sem, rpw)

    return k


NSPLIT = 8
BH = B // NSPLIT
_kth_sc_half = _make_kth_sc(BH * N)


@jax.jit
def kernel(x, W, bias):
    b, c, h, w = x.shape
    n = h * w
    xr = x.reshape(b, c, n)
    brow = bias.reshape(2 * c, 1)

    feat, pos = pl.pallas_call(
        _featpos_body,
        grid=(b, n // NT_PROJ),
        in_specs=[
            pl.BlockSpec((1, c, NT_PROJ), lambda i, j: (i, 0, j)),
            pl.BlockSpec((2 * c, c), lambda i, j: (0, 0)),
            pl.BlockSpec((2 * c, 1), lambda i, j: (0, 0)),
        ],
        out_specs=[
            pl.BlockSpec((1, c, NT_PROJ), lambda i, j: (i, 0, j)),
            pl.BlockSpec((1, c, NT_PROJ), lambda i, j: (i, 0, j)),
        ],
        out_shape=[
            jax.ShapeDtypeStruct((b, c, n), jnp.float32),
            jax.ShapeDtypeStruct((b, c, n), jnp.float32),
        ],
    )(xr, W, brow)

    outs = []
    for hh in range(NSPLIT):
        pos_h = pos[hh * BH:(hh + 1) * BH]
        feat_h = feat[hh * BH:(hh + 1) * BH]
        sim_h = pl.pallas_call(
            _sim_body,
            grid=(BH, n // T_AGG),
            in_specs=[
                pl.BlockSpec((1, c, T_AGG), lambda i, j: (i, 0, j)),
                pl.BlockSpec((1, c, n), lambda i, j: (i, 0, 0)),
            ],
            out_specs=pl.BlockSpec((1, T_AGG, n), lambda i, j: (i, j, 0)),
            out_shape=jax.ShapeDtypeStruct((BH, n, n), jnp.float32),
        )(pos_h, pos_h)

        thr_h = _kth_sc_half(sim_h.reshape(BH * N, N))
        thr3_h = thr_h.reshape(BH * n // T_AGG, 1, T_AGG)

        out_h = pl.pallas_call(
            _agg_body,
            grid=(BH, n // T_AGG),
            in_specs=[
                pl.BlockSpec((1, T_AGG, n), lambda i, j: (i, j, 0)),
                pl.BlockSpec((1, 1, T_AGG),
                             lambda i, j: (i * (N // T_AGG) + j, 0, 0)),
                pl.BlockSpec((1, c, n), lambda i, j: (i, 0, 0)),
            ],
            out_specs=pl.BlockSpec((1, c, T_AGG), lambda i, j: (i, 0, j)),
            out_shape=jax.ShapeDtypeStruct((BH, c, n), jnp.float32),
        )(sim_h, thr3_h, feat_h)
        outs.append(out_h)

    out = jnp.concatenate(outs, axis=0)
    return out.reshape(b, c, h, w)
